# baseline (speedup 1.0000x reference)
import functools

import jax
import jax.numpy as jnp
from jax import lax
from jax.experimental import pallas as pl
from jax.experimental.pallas import tpu as pltpu
from jax.experimental.pallas import tpu_sc as plsc

_B = 16384
_V = 10
_D = 32
_DP = 128

_cached = {}


def _make_kernel():
    if "k" in _cached:
        return _cached["k"]
    info = plsc.get_sparse_core_info()
    nc, ns, nl = info.num_cores, info.num_subcores, info.num_lanes
    nw = nc * ns
    b_per_w = _B // nw
    n_lines = b_per_w // 2
    n_h = 2
    lines_h = n_lines // n_h
    mesh = plsc.VectorSubcoreMesh(core_axis_name="c", subcore_axis_name="s")

    @functools.partial(
        pl.kernel,
        mesh=mesh,
        out_type=jax.ShapeDtypeStruct((_B, _D), jnp.float32),
        scratch_types=[
            pltpu.VMEM((b_per_w,), jnp.int32),
            pltpu.VMEM((n_lines,), jnp.int32),
            pltpu.VMEM((_V, _D), jnp.float32),
            pltpu.VMEM((_V * _V, _DP), jnp.float32),
            pltpu.VMEM_SHARED((_V * _V, _DP), jnp.float32),
            pltpu.VMEM((n_h, lines_h, _DP), jnp.float32),
            pltpu.VMEM((b_per_w, _D), jnp.float32),
            pltpu.SemaphoreType.DMA,
            pltpu.SemaphoreType.DMA,
            pltpu.SemaphoreType.DMA,
            pltpu.SemaphoreType.DMA,
        ],
        compiler_params=pltpu.CompilerParams(needs_layout_passes=False),
    )
    def k(t_hbm, table_hbm, out_hbm, idx_v, pidx_v, table_v, ppad_v, ppad_sh,
          lines_v, out_v, sem_i, sem_g0, sem_g1, sem_o):
        sid = lax.axis_index("s")
        wid = sid * nc + lax.axis_index("c")
        base = wid * b_per_w
        cp_i = pltpu.async_copy(t_hbm.at[pl.ds(base, b_per_w)], idx_v, sem_i)

        @pl.when(sid == 0)
        def _stage_pairs():
            pltpu.sync_copy(table_hbm, table_v)
            for a in range(_V):
                lo = table_v[a, pl.ds(0, nl)]
                hi = table_v[a, pl.ds(nl, nl)]
                for b in range(_V):
                    ppad_v[a * _V + b, pl.ds(0, nl)] = lo
                    ppad_v[a * _V + b, pl.ds(nl, nl)] = hi
                    ppad_v[b * _V + a, pl.ds(2 * nl, nl)] = lo
                    ppad_v[b * _V + a, pl.ds(3 * nl, nl)] = hi
            pltpu.sync_copy(ppad_v, ppad_sh)

        plsc.subcore_barrier()
        cp_i.wait()
        for v in range(n_lines // nl):
            a = idx_v[pl.ds(v * nl, nl)]
            b = idx_v[pl.ds(n_lines + v * nl, nl)]
            pidx_v[pl.ds(v * nl, nl)] = a * _V + b

        g_sems = [sem_g0, sem_g1]
        cps_g = [
            pltpu.async_copy(
                ppad_sh.at[pidx_v.at[pl.ds(h * lines_h, lines_h)]],
                lines_v.at[h],
                g_sems[h],
            )
            for h in range(n_h)
        ]
        outcps = []
        for h in range(n_h):
            cps_g[h].wait()

            def body(i, carry, h=h):
                ra = h * lines_h + i
                rb = n_lines + ra
                out_v[ra, pl.ds(0, nl)] = lines_v[h, i, pl.ds(0, nl)]
                out_v[ra, pl.ds(nl, nl)] = lines_v[h, i, pl.ds(nl, nl)]
                out_v[rb, pl.ds(0, nl)] = lines_v[h, i, pl.ds(2 * nl, nl)]
                out_v[rb, pl.ds(nl, nl)] = lines_v[h, i, pl.ds(3 * nl, nl)]
                return carry

            lax.fori_loop(0, lines_h, body, 0)
            outcps.append(
                pltpu.async_copy(
                    out_v.at[pl.ds(h * lines_h, lines_h)],
                    out_hbm.at[pl.ds(base + h * lines_h, lines_h)],
                    sem_o,
                )
            )
            outcps.append(
                pltpu.async_copy(
                    out_v.at[pl.ds(n_lines + h * lines_h, lines_h)],
                    out_hbm.at[pl.ds(base + n_lines + h * lines_h, lines_h)],
                    sem_o,
                )
            )
        for cp in outcps:
            cp.wait()

    _cached["k"] = k
    return k


def kernel(t, table):
    k = _make_kernel()
    return k(t.astype(jnp.int32), table.astype(jnp.float32))
